# Initial kernel scaffold; baseline (speedup 1.0000x reference)
import jax
import jax.numpy as jnp
from jax.experimental import pallas as pl


def kernel(x_v, edge_index_v, edge_attr_v, ptr_v, x_p, edge_index_p, edge_attr_p, ptr_p, params):
    raise NotImplementedError("write your pallas kernel here")


# speedup vs baseline: 1.6937x; 1.6937x over previous
import functools

import jax
import jax.numpy as jnp
from jax import lax
from jax.experimental import pallas as pl
from jax.experimental.pallas import tpu as pltpu
from jax.experimental.pallas import tpu_sc as plsc

N = 10000
E = 160000
EP = 163840
NC = 2
NS = 16
NT = NC * NS
EPT = EP // NT
B = 32
NBLK = EPT // B
NP = 10240
RPT = NP // NS
BM = 1000


def _gauss_all(ea, w1, w2, bb):
    g = w1.shape[1]
    bm = 8000

    def body(a_ref, w1_ref, w2_ref, b_ref, o_ref):
        a = a_ref[...]
        z = (jnp.dot(a * a, w1_ref[...], preferred_element_type=jnp.float32)
             + jnp.dot(a, w2_ref[...], preferred_element_type=jnp.float32)
             + b_ref[...])
        o_ref[...] = jnp.exp(z)

    return pl.pallas_call(
        body,
        grid=(E // bm,),
        in_specs=[
            pl.BlockSpec((bm, 3), lambda r: (r, 0)),
            pl.BlockSpec((3, g), lambda r: (0, 0)),
            pl.BlockSpec((3, g), lambda r: (0, 0)),
            pl.BlockSpec((1, g), lambda r: (0, 0)),
        ],
        out_specs=pl.BlockSpec((bm, g), lambda r: (r, 0)),
        out_shape=jax.ShapeDtypeStruct((E, g), jnp.float32),
    )(ea, w1, w2, bb)


def _mm_xg(x, wp, nch, kc):
    cin = x.shape[1]

    def body(x_ref, w_ref, o_ref):
        o_ref[0] = jnp.dot(x_ref[...], w_ref[...],
                           preferred_element_type=jnp.float32)

    return pl.pallas_call(
        body,
        grid=(N // BM, nch),
        in_specs=[
            pl.BlockSpec((BM, cin), lambda r, c: (r, 0)),
            pl.BlockSpec((cin, kc), lambda r, c: (0, c)),
        ],
        out_specs=pl.BlockSpec((1, BM, kc), lambda r, c: (c, r, 0)),
        out_shape=jax.ShapeDtypeStruct((nch, N, kc), jnp.float32),
    )(x, wp)


def _conv_out(aggf, cntp, x, rootp, biasp, nch, c, cg):
    cin = x.shape[1]

    def body(a_ref, c_ref, x_ref, r_ref, b_ref, o_ref):
        cnt = c_ref[0, :, 0:1] + c_ref[1, :, 0:1]
        inv = 1.0 / jnp.maximum(cnt, 1.0)
        xb = x_ref[...]
        for ch in range(nch):
            agg = a_ref[0, ch] + a_ref[1, ch]
            if cg != c:
                agg = agg[:, :c]
            o_ref[:, ch, :] = (agg * inv
                               + jnp.dot(xb, r_ref[:, ch, :],
                                         preferred_element_type=jnp.float32)
                               + b_ref[0, ch])

    out = pl.pallas_call(
        body,
        grid=(N // BM,),
        in_specs=[
            pl.BlockSpec((2, nch, BM, cg), lambda r: (0, 0, r, 0)),
            pl.BlockSpec((2, BM, 128), lambda r: (0, r, 0)),
            pl.BlockSpec((BM, cin), lambda r: (r, 0)),
            pl.BlockSpec((cin, nch, c), lambda r: (0, 0, 0)),
            pl.BlockSpec((1, nch, c), lambda r: (0, 0, 0)),
        ],
        out_specs=pl.BlockSpec((BM, nch, c), lambda r: (r, 0, 0)),
        out_shape=jax.ShapeDtypeStruct((N, nch, c), jnp.float32),
    )(aggf, cntp, x, rootp, biasp)
    return out.reshape(N, nch * c)


def _bn_stats(o):
    c = o.shape[1]

    def body(x_ref, s_ref):
        @pl.when(pl.program_id(0) == 0)
        def _():
            s_ref[...] = jnp.zeros_like(s_ref)

        x = x_ref[...]
        s_ref[0:1, :] += jnp.sum(x, 0, keepdims=True)
        s_ref[1:2, :] += jnp.sum(x * x, 0, keepdims=True)

    return pl.pallas_call(
        body,
        grid=(N // BM,),
        in_specs=[pl.BlockSpec((BM, c), lambda r: (r, 0))],
        out_specs=pl.BlockSpec((2, c), lambda r: (0, 0)),
        out_shape=jax.ShapeDtypeStruct((2, c), jnp.float32),
    )(o)


def _bn_apply(o, bn, res=None, act=None):
    c = o.shape[1]
    stats = _bn_stats(o)
    ins = [o, stats, bn["gamma"].reshape(1, c), bn["beta"].reshape(1, c)]
    specs = [
        pl.BlockSpec((BM, c), lambda r: (r, 0)),
        pl.BlockSpec((2, c), lambda r: (0, 0)),
        pl.BlockSpec((1, c), lambda r: (0, 0)),
        pl.BlockSpec((1, c), lambda r: (0, 0)),
    ]
    if res is not None:
        ins.append(res)
        specs.append(pl.BlockSpec((BM, c), lambda r: (r, 0)))

    def body(*refs):
        x_ref, s_ref, g_ref, b_ref = refs[:4]
        o_ref = refs[-1]
        m = s_ref[0:1, :] * (1.0 / N)
        var = s_ref[1:2, :] * (1.0 / N) - m * m
        y = (x_ref[...] - m) * lax.rsqrt(var + 1e-5) * g_ref[...] + b_ref[...]
        if res is not None:
            y = y + refs[4][...]
        if act == "elu":
            y = jnp.where(y > 0, y, jnp.exp(jnp.minimum(y, 0.0)) - 1.0)
        o_ref[...] = y

    return pl.pallas_call(
        body,
        grid=(N // BM,),
        in_specs=specs,
        out_specs=pl.BlockSpec((BM, c), lambda r: (r, 0)),
        out_shape=jax.ShapeDtypeStruct((N, c), jnp.float32),
    )(*ins)


def _head(hv8, hp8, w1, b1, gam, bet, w2, b2):
    def body(a_ref, p_ref, w1_ref, b1_ref, g_ref, be_ref, w2_ref, b2_ref,
             o_ref):
        x = jnp.concatenate([a_ref[...], p_ref[...]], axis=1)
        h = jnp.dot(x, w1_ref[...], preferred_element_type=jnp.float32) \
            + b1_ref[...]
        h = jnp.where(h > 0, h, jnp.exp(jnp.minimum(h, 0.0)) - 1.0)
        m = jnp.mean(h, 0, keepdims=True)
        v = jnp.mean(h * h, 0, keepdims=True) - m * m
        h = (h - m) * lax.rsqrt(v + 1e-5) * g_ref[...] + be_ref[...]
        z = jnp.dot(h, w2_ref[...], preferred_element_type=jnp.float32) \
            + b2_ref[...]
        mx = jnp.max(z, 1, keepdims=True)
        lse = jnp.log(jnp.sum(jnp.exp(z - mx), 1, keepdims=True)) + mx
        o_ref[...] = z - lse

    nclass = w2.shape[1]
    return pl.pallas_call(
        body,
        out_shape=jax.ShapeDtypeStruct((8, nclass), jnp.float32),
    )(hv8, hp8, w1, b1.reshape(1, -1), gam.reshape(1, -1),
      bet.reshape(1, -1), w2, b2.reshape(1, -1))


@functools.cache
def _pair_kernel():
    mesh = plsc.VectorSubcoreMesh(core_axis_name="c", subcore_axis_name="s")

    @functools.partial(
        pl.kernel,
        mesh=mesh,
        out_type=jax.ShapeDtypeStruct((4 * NP, 128), jnp.float32),
        scratch_types=[
            pltpu.VMEM((B,), jnp.int32),
            pltpu.VMEM((B,), jnp.int32),
            pltpu.VMEM((5, B), jnp.int32),
            pltpu.VMEM((B, 16), jnp.float32),
            pltpu.VMEM((5, B, 128), jnp.float32),
            pltpu.VMEM((B, 128), jnp.float32),
            pltpu.VMEM_SHARED((NP, 128), jnp.float32),
            pltpu.SemaphoreType.DMA,
        ],
    )
    def kern(xg5, xg1, srcr, dstr, gaussr, zr, out, srcv, dstv, idxv,
             gaussv, gbuf, msg, spmem, sem):
        p = lax.axis_index("c")
        s = lax.axis_index("s")
        wid = s * NC + p
        ebase = wid * EPT
        for ph in range(2):
            pltpu.sync_copy(zr, spmem.at[pl.ds(s * RPT, RPT)])
            plsc.subcore_barrier()

            def bbody(b, carry):
                e0 = ebase + b * B
                pltpu.sync_copy(srcr.at[pl.ds(e0, B)], srcv)
                pltpu.sync_copy(dstr.at[pl.ds(e0, B)], dstv)
                pltpu.sync_copy(gaussr.at[pl.ds(e0, B)], gaussv)
                nk = 5 if ph == 0 else 1
                for kk in range(nk):
                    for i in range(B // 16):
                        sl = pl.ds(i * 16, 16)
                        if ph == 0:
                            idxv[kk, sl] = srcv[sl] * 5 + kk
                        else:
                            idxv[kk, sl] = srcv[sl]
                src_tab = xg5 if ph == 0 else xg1
                descs = [
                    pltpu.async_copy(src_tab.at[idxv.at[kk]],
                                     gbuf.at[kk], sem)
                    for kk in range(nk)
                ]
                for d in descs:
                    d.wait()

                def ebody(e, c2):
                    gv = gaussv[e, :]
                    for j in range(8):
                        sl = pl.ds(j * 16, 16)
                        if ph == 0:
                            acc = gv[0] * gbuf[0, e, sl]
                            for kk in range(1, 5):
                                acc = acc + gv[kk] * gbuf[kk, e, sl]
                        else:
                            acc = gv[5] * gbuf[0, e, sl]
                        msg[e, sl] = acc
                    return c2

                lax.fori_loop(0, B, ebody, 0)
                pltpu.sync_copy(msg, spmem.at[dstv], add=True)
                return carry

            lax.fori_loop(0, NBLK, bbody, 0)
            plsc.subcore_barrier()
            off = (ph * 2 + p) * NP + s * RPT
            pltpu.sync_copy(spmem.at[pl.ds(s * RPT, RPT)],
                            out.at[pl.ds(off, RPT)])
            plsc.subcore_barrier()

    return kern


def _pair_call(xg5, xg1, srcp, dstp, g16):
    zr = jnp.zeros((RPT, 128), jnp.float32)
    out = _pair_kernel()(xg5, xg1, srcp, dstp, g16, zr)
    return out.reshape(2, 2, NP, 128)


def _dummy1():
    return jnp.zeros((N, 128), jnp.float32)


def _xg_chunks(x, p, k):
    cin, cout = p["root"].shape
    c = min(cout, 128)
    nch = cout // c
    g4 = p["g"].reshape(cin, k, nch, c)
    if c != 128:
        g4 = jnp.pad(g4, ((0, 0), (0, 0), (0, 0), (0, 128 - c)))
    wp = g4.transpose(0, 2, 1, 3).reshape(cin, nch * k * 128)
    xgc = _mm_xg(x, wp, nch, k * 128)
    tabs = [xgc[ch].reshape(N * k, 128) for ch in range(nch)]
    return tabs, nch, c


def _conv_from_aggs(aggs, cntp, x, p, nch, c):
    cin = p["root"].shape[0]
    aggf = jnp.stack(aggs, axis=1)
    rootp = p["root"].reshape(cin, nch, c)
    biasp = p["bias"].reshape(1, nch, c)
    return _conv_out(aggf, cntp, x, rootp, biasp, nch, c, 128)


def _gmm_conv5(x, srcp, dstp, g16, cntp, p):
    tabs, nch, c = _xg_chunks(x, p, 5)
    aggs = [_pair_call(t, _dummy1(), srcp, dstp, g16)[0] for t in tabs]
    return _conv_from_aggs(aggs, cntp, x, p, nch, c)


def _gauss_weights(plist):
    w1, w2, bb = [], [], []
    for p in plist:
        mu, sigma = p["mu"], p["sigma"]
        inv = 1.0 / (1e-15 + sigma * sigma)
        w1.append((-0.5 * inv).T)
        w2.append((mu * inv).T)
        bb.append(-0.5 * jnp.sum(mu * mu * inv, axis=1))
    return (jnp.concatenate(w1, 1), jnp.concatenate(w2, 1),
            jnp.concatenate(bb)[None, :])


def _g16(cols):
    return jnp.pad(cols, ((0, 0), (0, 16 - cols.shape[1])))


def _res_block(x, srcp, dstp, g_a, g_c2, cntp, p):
    tabs1, nch, c = _xg_chunks(x, p["c1"], 5)
    tabs_s, _, _ = _xg_chunks(x, p["sc"], 1)
    agg_a, agg_b = [], []
    for t5, t1 in zip(tabs1, tabs_s):
        o4 = _pair_call(t5, t1, srcp, dstp, g_a)
        agg_a.append(o4[0])
        agg_b.append(o4[1])
    o1 = _conv_from_aggs(agg_a, cntp, x, p["c1"], nch, c)
    os = _conv_from_aggs(agg_b, cntp, x, p["sc"], nch, c)
    h = _bn_apply(o1, p["bn1"], act="elu")
    o2 = _gmm_conv5(h, srcp, dstp, g_c2, cntp, p["c2"])
    sbn = _bn_apply(os, p["bns"])
    return _bn_apply(o2, p["bn2"], res=sbn, act="elu")


def _stream(x0, ei, ea, p_conv, p_bn, blocks):
    srcp = jnp.pad(ei[0], (0, EP - E)).astype(jnp.int32)
    dstp = jnp.pad(ei[1], (0, EP - E)).astype(jnp.int32)
    plist = [p_conv]
    for b in blocks:
        plist += [b["c1"], b["c2"], b["sc"]]
    w1, w2, bb = _gauss_weights(plist)
    gall = _gauss_all(ea, w1, w2, bb)
    gpad = jnp.pad(gall, ((0, EP - E), (0, 0)))
    valid = (jnp.arange(EP) < E).astype(jnp.float32)[:, None]
    g_conv = _g16(jnp.concatenate([gpad[:, 0:5], valid], axis=1))
    ones_tab = jnp.ones((N, 128), jnp.float32)
    tabs0, nch0, c0 = _xg_chunks(x0, p_conv, 5)
    o4 = _pair_call(tabs0[0], ones_tab, srcp, dstp, g_conv)
    cntp = o4[1]
    o0 = _conv_from_aggs([o4[0]], cntp, x0, p_conv, nch0, c0)
    h = _bn_apply(o0, p_bn, act="elu")
    goff = 5
    for b in blocks:
        g_a = _g16(jnp.concatenate(
            [gpad[:, goff:goff + 5], gpad[:, goff + 10:goff + 11]], axis=1))
        g_c2 = _g16(gpad[:, goff + 5:goff + 10])
        h = _res_block(h, srcp, dstp, g_a, g_c2, cntp, b)
        goff += 11
    return h


def kernel(x_v, edge_index_v, edge_attr_v, ptr_v, x_p, edge_index_p,
           edge_attr_p, ptr_p, params):
    pr = params
    hv = _stream(x_v, edge_index_v, edge_attr_v, pr["conv_v"], pr["bn_v"],
                 [pr["block1"], pr["block2"], pr["block3"]])
    hp = _stream(x_p, edge_index_p, edge_attr_p, pr["conv_p"], pr["bn_p"],
                 [pr["block4"], pr["block5"], pr["block6"]])
    vi_v = ptr_v[1:] - 1
    vi_p = ptr_p[1:] - 1
    return _head(hv[vi_v], hp[vi_p], pr["fc1_w"], pr["fc1_b"],
                 pr["bn_fc"]["gamma"], pr["bn_fc"]["beta"],
                 pr["fc2_w"], pr["fc2_b"])


# speedup vs baseline: 1.8442x; 1.0888x over previous
import functools

import jax
import jax.numpy as jnp
from jax import lax
from jax.experimental import pallas as pl
from jax.experimental.pallas import tpu as pltpu
from jax.experimental.pallas import tpu_sc as plsc

N = 10000
E = 160000
EP = 163840
NC = 2
NS = 16
NT = NC * NS
EPT = EP // NT
B = 32
NBLK = EPT // B
SB = 256
NP = 10240
RPT = NP // NS
BM = 1000


def _gauss_all(ea, w1, w2, bb):
    g = w1.shape[1]
    bm = 8000

    def body(a_ref, w1_ref, w2_ref, b_ref, o_ref):
        a = a_ref[...]
        z = (jnp.dot(a * a, w1_ref[...], preferred_element_type=jnp.float32)
             + jnp.dot(a, w2_ref[...], preferred_element_type=jnp.float32)
             + b_ref[...])
        o_ref[...] = jnp.exp(z)

    return pl.pallas_call(
        body,
        grid=(E // bm,),
        in_specs=[
            pl.BlockSpec((bm, 3), lambda r: (r, 0)),
            pl.BlockSpec((3, g), lambda r: (0, 0)),
            pl.BlockSpec((3, g), lambda r: (0, 0)),
            pl.BlockSpec((1, g), lambda r: (0, 0)),
        ],
        out_specs=pl.BlockSpec((bm, g), lambda r: (r, 0)),
        out_shape=jax.ShapeDtypeStruct((E, g), jnp.float32),
    )(ea, w1, w2, bb)


def _mm_xg(x, wp, nch, kc):
    cin = x.shape[1]

    def body(x_ref, w_ref, o_ref):
        o_ref[0] = jnp.dot(x_ref[...], w_ref[...],
                           preferred_element_type=jnp.float32)

    return pl.pallas_call(
        body,
        grid=(N // BM, nch),
        in_specs=[
            pl.BlockSpec((BM, cin), lambda r, c: (r, 0)),
            pl.BlockSpec((cin, kc), lambda r, c: (0, c)),
        ],
        out_specs=pl.BlockSpec((1, BM, kc), lambda r, c: (c, r, 0)),
        out_shape=jax.ShapeDtypeStruct((nch, N, kc), jnp.float32),
    )(x, wp)


def _conv_out(aggf, cntp, x, rootp, biasp, nch, c, cg):
    cin = x.shape[1]

    def body(a_ref, c_ref, x_ref, r_ref, b_ref, o_ref):
        cnt = c_ref[0, :, 0:1] + c_ref[1, :, 0:1]
        inv = 1.0 / jnp.maximum(cnt, 1.0)
        xb = x_ref[...]
        for ch in range(nch):
            agg = a_ref[0, ch] + a_ref[1, ch]
            if cg != c:
                agg = agg[:, :c]
            o_ref[:, ch, :] = (agg * inv
                               + jnp.dot(xb, r_ref[:, ch, :],
                                         preferred_element_type=jnp.float32)
                               + b_ref[0, ch])

    out = pl.pallas_call(
        body,
        grid=(N // BM,),
        in_specs=[
            pl.BlockSpec((2, nch, BM, cg), lambda r: (0, 0, r, 0)),
            pl.BlockSpec((2, BM, 128), lambda r: (0, r, 0)),
            pl.BlockSpec((BM, cin), lambda r: (r, 0)),
            pl.BlockSpec((cin, nch, c), lambda r: (0, 0, 0)),
            pl.BlockSpec((1, nch, c), lambda r: (0, 0, 0)),
        ],
        out_specs=pl.BlockSpec((BM, nch, c), lambda r: (r, 0, 0)),
        out_shape=jax.ShapeDtypeStruct((N, nch, c), jnp.float32),
    )(aggf, cntp, x, rootp, biasp)
    return out.reshape(N, nch * c)


def _bn_stats(o):
    c = o.shape[1]

    def body(x_ref, s_ref):
        @pl.when(pl.program_id(0) == 0)
        def _():
            s_ref[...] = jnp.zeros_like(s_ref)

        x = x_ref[...]
        s_ref[0:1, :] += jnp.sum(x, 0, keepdims=True)
        s_ref[1:2, :] += jnp.sum(x * x, 0, keepdims=True)

    return pl.pallas_call(
        body,
        grid=(N // BM,),
        in_specs=[pl.BlockSpec((BM, c), lambda r: (r, 0))],
        out_specs=pl.BlockSpec((2, c), lambda r: (0, 0)),
        out_shape=jax.ShapeDtypeStruct((2, c), jnp.float32),
    )(o)


def _bn_apply(o, bn, res=None, act=None):
    c = o.shape[1]
    stats = _bn_stats(o)
    ins = [o, stats, bn["gamma"].reshape(1, c), bn["beta"].reshape(1, c)]
    specs = [
        pl.BlockSpec((BM, c), lambda r: (r, 0)),
        pl.BlockSpec((2, c), lambda r: (0, 0)),
        pl.BlockSpec((1, c), lambda r: (0, 0)),
        pl.BlockSpec((1, c), lambda r: (0, 0)),
    ]
    if res is not None:
        ins.append(res)
        specs.append(pl.BlockSpec((BM, c), lambda r: (r, 0)))

    def body(*refs):
        x_ref, s_ref, g_ref, b_ref = refs[:4]
        o_ref = refs[-1]
        m = s_ref[0:1, :] * (1.0 / N)
        var = s_ref[1:2, :] * (1.0 / N) - m * m
        y = (x_ref[...] - m) * lax.rsqrt(var + 1e-5) * g_ref[...] + b_ref[...]
        if res is not None:
            y = y + refs[4][...]
        if act == "elu":
            y = jnp.where(y > 0, y, jnp.exp(jnp.minimum(y, 0.0)) - 1.0)
        o_ref[...] = y

    return pl.pallas_call(
        body,
        grid=(N // BM,),
        in_specs=specs,
        out_specs=pl.BlockSpec((BM, c), lambda r: (r, 0)),
        out_shape=jax.ShapeDtypeStruct((N, c), jnp.float32),
    )(*ins)


def _head(hv8, hp8, w1, b1, gam, bet, w2, b2):
    def body(a_ref, p_ref, w1_ref, b1_ref, g_ref, be_ref, w2_ref, b2_ref,
             o_ref):
        x = jnp.concatenate([a_ref[...], p_ref[...]], axis=1)
        h = jnp.dot(x, w1_ref[...], preferred_element_type=jnp.float32) \
            + b1_ref[...]
        h = jnp.where(h > 0, h, jnp.exp(jnp.minimum(h, 0.0)) - 1.0)
        m = jnp.mean(h, 0, keepdims=True)
        v = jnp.mean(h * h, 0, keepdims=True) - m * m
        h = (h - m) * lax.rsqrt(v + 1e-5) * g_ref[...] + be_ref[...]
        z = jnp.dot(h, w2_ref[...], preferred_element_type=jnp.float32) \
            + b2_ref[...]
        mx = jnp.max(z, 1, keepdims=True)
        lse = jnp.log(jnp.sum(jnp.exp(z - mx), 1, keepdims=True)) + mx
        o_ref[...] = z - lse

    nclass = w2.shape[1]
    return pl.pallas_call(
        body,
        out_shape=jax.ShapeDtypeStruct((8, nclass), jnp.float32),
    )(hv8, hp8, w1, b1.reshape(1, -1), gam.reshape(1, -1),
      bet.reshape(1, -1), w2, b2.reshape(1, -1))


@functools.cache
def _pair_kernel():
    mesh = plsc.VectorSubcoreMesh(core_axis_name="c", subcore_axis_name="s")

    @functools.partial(
        pl.kernel,
        mesh=mesh,
        out_type=jax.ShapeDtypeStruct((4 * NP, 128), jnp.float32),
        scratch_types=[
            pltpu.VMEM((SB // B, B), jnp.int32),
            pltpu.VMEM((SB // B, B), jnp.int32),
            pltpu.VMEM((SB * 16,), jnp.float32),
            pltpu.VMEM((B, 640), jnp.float32),
            pltpu.VMEM((B, 128), jnp.float32),
            pltpu.VMEM_SHARED((NP, 128), jnp.float32),
            pltpu.SemaphoreType.DMA,
        ],
    )
    def kern(xg5, xg1, src2, dst2, gaussr, zr, out, srcv2, dstv2,
             gaussv, gbuf, msg, spmem, sem):
        p = lax.axis_index("c")
        s = lax.axis_index("s")
        wid = s * NC + p
        ebase = wid * EPT
        nsb = SB // B
        for ph in range(2):
            pltpu.sync_copy(zr, spmem.at[pl.ds(s * RPT, RPT)])
            plsc.subcore_barrier()

            def sbody(sb, carry):
                e0 = pl.multiple_of(ebase + sb * SB, SB)
                r0 = pl.multiple_of(e0 // B, SB // B)
                pltpu.sync_copy(src2.at[pl.ds(r0, nsb)], srcv2)
                pltpu.sync_copy(dst2.at[pl.ds(r0, nsb)], dstv2)
                g0 = pl.multiple_of(e0 * 16, SB * 16)
                pltpu.sync_copy(gaussr.at[pl.ds(g0, SB * 16)], gaussv)

                def jbody(j, c1):
                    if ph == 0:
                        pltpu.async_copy(xg5.at[srcv2.at[j]], gbuf,
                                         sem).wait()
                    else:
                        pltpu.async_copy(xg1.at[srcv2.at[j]], msg,
                                         sem).wait()

                    def ebody(e, c2):
                        gv = gaussv[pl.ds((j * B + e) * 16, 16)]
                        for jj in range(8):
                            sl = pl.ds(jj * 16, 16)
                            if ph == 0:
                                acc = gv[0] * gbuf[e, sl]
                                for kk in range(1, 5):
                                    acc = acc + gv[kk] * gbuf[
                                        e, pl.ds(kk * 128 + jj * 16, 16)]
                                msg[e, sl] = acc
                            else:
                                msg[e, sl] = gv[5] * msg[e, sl]
                        return c2

                    lax.fori_loop(0, B, ebody, 0)
                    pltpu.sync_copy(msg, spmem.at[dstv2.at[j]], add=True)
                    return c1

                lax.fori_loop(0, nsb, jbody, 0)
                return carry

            lax.fori_loop(0, EPT // SB, sbody, 0)
            plsc.subcore_barrier()
            off = (ph * 2 + p) * NP + s * RPT
            pltpu.sync_copy(spmem.at[pl.ds(s * RPT, RPT)],
                            out.at[pl.ds(off, RPT)])
            plsc.subcore_barrier()

    return kern


def _pair_call(xg5, xg1, srcp, dstp, g16):
    zr = jnp.zeros((RPT, 128), jnp.float32)
    src2 = srcp.reshape(EP // B, B)
    dst2 = dstp.reshape(EP // B, B)
    out = _pair_kernel()(xg5, xg1, src2, dst2, g16.reshape(EP * 16), zr)
    return out.reshape(2, 2, NP, 128)


def _dummy1():
    return jnp.zeros((N, 128), jnp.float32)


def _xg_chunks(x, p, k):
    cin, cout = p["root"].shape
    c = min(cout, 128)
    nch = cout // c
    g4 = p["g"].reshape(cin, k, nch, c)
    if c != 128:
        g4 = jnp.pad(g4, ((0, 0), (0, 0), (0, 0), (0, 128 - c)))
    wp = g4.transpose(0, 2, 1, 3).reshape(cin, nch * k * 128)
    xgc = _mm_xg(x, wp, nch, k * 128)
    tabs = [xgc[ch] for ch in range(nch)]
    return tabs, nch, c


def _conv_from_aggs(aggs, cntp, x, p, nch, c):
    cin = p["root"].shape[0]
    aggf = jnp.stack(aggs, axis=1)
    rootp = p["root"].reshape(cin, nch, c)
    biasp = p["bias"].reshape(1, nch, c)
    return _conv_out(aggf, cntp, x, rootp, biasp, nch, c, 128)


def _gmm_conv5(x, srcp, dstp, g16, cntp, p):
    tabs, nch, c = _xg_chunks(x, p, 5)
    aggs = [_pair_call(t, _dummy1(), srcp, dstp, g16)[0] for t in tabs]
    return _conv_from_aggs(aggs, cntp, x, p, nch, c)


def _gauss_weights(plist):
    w1, w2, bb = [], [], []
    for p in plist:
        mu, sigma = p["mu"], p["sigma"]
        inv = 1.0 / (1e-15 + sigma * sigma)
        w1.append((-0.5 * inv).T)
        w2.append((mu * inv).T)
        bb.append(-0.5 * jnp.sum(mu * mu * inv, axis=1))
    return (jnp.concatenate(w1, 1), jnp.concatenate(w2, 1),
            jnp.concatenate(bb)[None, :])


def _g16(cols):
    return jnp.pad(cols, ((0, 0), (0, 16 - cols.shape[1])))


def _res_block(x, srcp, dstp, g_a, g_c2, cntp, p):
    tabs1, nch, c = _xg_chunks(x, p["c1"], 5)
    tabs_s, _, _ = _xg_chunks(x, p["sc"], 1)
    agg_a, agg_b = [], []
    for t5, t1 in zip(tabs1, tabs_s):
        o4 = _pair_call(t5, t1, srcp, dstp, g_a)
        agg_a.append(o4[0])
        agg_b.append(o4[1])
    o1 = _conv_from_aggs(agg_a, cntp, x, p["c1"], nch, c)
    os = _conv_from_aggs(agg_b, cntp, x, p["sc"], nch, c)
    h = _bn_apply(o1, p["bn1"], act="elu")
    o2 = _gmm_conv5(h, srcp, dstp, g_c2, cntp, p["c2"])
    sbn = _bn_apply(os, p["bns"])
    return _bn_apply(o2, p["bn2"], res=sbn, act="elu")


def _stream(x0, ei, ea, p_conv, p_bn, blocks):
    srcp = jnp.pad(ei[0], (0, EP - E)).astype(jnp.int32)
    dstp = jnp.pad(ei[1], (0, EP - E)).astype(jnp.int32)
    plist = [p_conv]
    for b in blocks:
        plist += [b["c1"], b["c2"], b["sc"]]
    w1, w2, bb = _gauss_weights(plist)
    gall = _gauss_all(ea, w1, w2, bb)
    gpad = jnp.pad(gall, ((0, EP - E), (0, 0)))
    valid = (jnp.arange(EP) < E).astype(jnp.float32)[:, None]
    g_conv = _g16(jnp.concatenate([gpad[:, 0:5], valid], axis=1))
    ones_tab = jnp.ones((N, 128), jnp.float32)
    tabs0, nch0, c0 = _xg_chunks(x0, p_conv, 5)
    o4 = _pair_call(tabs0[0], ones_tab, srcp, dstp, g_conv)
    cntp = o4[1]
    o0 = _conv_from_aggs([o4[0]], cntp, x0, p_conv, nch0, c0)
    h = _bn_apply(o0, p_bn, act="elu")
    goff = 5
    for b in blocks:
        g_a = _g16(jnp.concatenate(
            [gpad[:, goff:goff + 5], gpad[:, goff + 10:goff + 11]], axis=1))
        g_c2 = _g16(gpad[:, goff + 5:goff + 10])
        h = _res_block(h, srcp, dstp, g_a, g_c2, cntp, b)
        goff += 11
    return h


def kernel(x_v, edge_index_v, edge_attr_v, ptr_v, x_p, edge_index_p,
           edge_attr_p, ptr_p, params):
    pr = params
    hv = _stream(x_v, edge_index_v, edge_attr_v, pr["conv_v"], pr["bn_v"],
                 [pr["block1"], pr["block2"], pr["block3"]])
    hp = _stream(x_p, edge_index_p, edge_attr_p, pr["conv_p"], pr["bn_p"],
                 [pr["block4"], pr["block5"], pr["block6"]])
    vi_v = ptr_v[1:] - 1
    vi_p = ptr_p[1:] - 1
    return _head(hv[vi_v], hp[vi_p], pr["fc1_w"], pr["fc1_b"],
                 pr["bn_fc"]["gamma"], pr["bn_fc"]["beta"],
                 pr["fc2_w"], pr["fc2_b"])


# speedup vs baseline: 2.2093x; 1.1980x over previous
import functools

import jax
import jax.numpy as jnp
from jax import lax
from jax.experimental import pallas as pl
from jax.experimental.pallas import tpu as pltpu
from jax.experimental.pallas import tpu_sc as plsc

N = 10000
E = 160000
EP = 163840
NC = 2
NS = 16
NT = NC * NS
EPT = EP // NT
B = 16
NBLK = EPT // B
SB = 256
NP = 10240
RPT = NP // NS
BM = 1000


def _gauss_all(ea, w1, w2, bb):
    g = w1.shape[1]
    bm = 8000

    def body(a_ref, w1_ref, w2_ref, b_ref, o_ref):
        a = a_ref[...]
        z = (jnp.dot(a * a, w1_ref[...], preferred_element_type=jnp.float32)
             + jnp.dot(a, w2_ref[...], preferred_element_type=jnp.float32)
             + b_ref[...])
        o_ref[...] = jnp.exp(z)

    return pl.pallas_call(
        body,
        grid=(E // bm,),
        in_specs=[
            pl.BlockSpec((bm, 3), lambda r: (r, 0)),
            pl.BlockSpec((3, g), lambda r: (0, 0)),
            pl.BlockSpec((3, g), lambda r: (0, 0)),
            pl.BlockSpec((1, g), lambda r: (0, 0)),
        ],
        out_specs=pl.BlockSpec((bm, g), lambda r: (r, 0)),
        out_shape=jax.ShapeDtypeStruct((E, g), jnp.float32),
    )(ea, w1, w2, bb)


def _mm_xg(x, wp, nch, kc):
    cin = x.shape[1]

    def body(x_ref, w_ref, o_ref):
        o_ref[0] = jnp.dot(x_ref[...], w_ref[...],
                           preferred_element_type=jnp.float32)

    return pl.pallas_call(
        body,
        grid=(N // BM, nch),
        in_specs=[
            pl.BlockSpec((BM, cin), lambda r, c: (r, 0)),
            pl.BlockSpec((cin, kc), lambda r, c: (0, c)),
        ],
        out_specs=pl.BlockSpec((1, BM, kc), lambda r, c: (c, r, 0)),
        out_shape=jax.ShapeDtypeStruct((nch, N, kc), jnp.float32),
    )(x, wp)


def _conv_out(aggf, cntp, x, rootp, biasp, nch, c, cg):
    cin = x.shape[1]

    def body(a_ref, c_ref, x_ref, r_ref, b_ref, o_ref):
        cnt = c_ref[0, :, 0:1] + c_ref[1, :, 0:1]
        inv = 1.0 / jnp.maximum(cnt, 1.0)
        xb = x_ref[...]
        for ch in range(nch):
            agg = a_ref[0, ch] + a_ref[1, ch]
            if cg != c:
                agg = agg[:, :c]
            o_ref[:, ch, :] = (agg * inv
                               + jnp.dot(xb, r_ref[:, ch, :],
                                         preferred_element_type=jnp.float32)
                               + b_ref[0, ch])

    out = pl.pallas_call(
        body,
        grid=(N // BM,),
        in_specs=[
            pl.BlockSpec((2, nch, BM, cg), lambda r: (0, 0, r, 0)),
            pl.BlockSpec((2, BM, 128), lambda r: (0, r, 0)),
            pl.BlockSpec((BM, cin), lambda r: (r, 0)),
            pl.BlockSpec((cin, nch, c), lambda r: (0, 0, 0)),
            pl.BlockSpec((1, nch, c), lambda r: (0, 0, 0)),
        ],
        out_specs=pl.BlockSpec((BM, nch, c), lambda r: (r, 0, 0)),
        out_shape=jax.ShapeDtypeStruct((N, nch, c), jnp.float32),
    )(aggf, cntp, x, rootp, biasp)
    return out.reshape(N, nch * c)


def _bn_stats(o):
    c = o.shape[1]

    def body(x_ref, s_ref):
        @pl.when(pl.program_id(0) == 0)
        def _():
            s_ref[...] = jnp.zeros_like(s_ref)

        x = x_ref[...]
        s_ref[0:1, :] += jnp.sum(x, 0, keepdims=True)
        s_ref[1:2, :] += jnp.sum(x * x, 0, keepdims=True)

    return pl.pallas_call(
        body,
        grid=(N // BM,),
        in_specs=[pl.BlockSpec((BM, c), lambda r: (r, 0))],
        out_specs=pl.BlockSpec((2, c), lambda r: (0, 0)),
        out_shape=jax.ShapeDtypeStruct((2, c), jnp.float32),
    )(o)


def _bn_apply(o, bn, res=None, act=None):
    c = o.shape[1]
    stats = _bn_stats(o)
    ins = [o, stats, bn["gamma"].reshape(1, c), bn["beta"].reshape(1, c)]
    specs = [
        pl.BlockSpec((BM, c), lambda r: (r, 0)),
        pl.BlockSpec((2, c), lambda r: (0, 0)),
        pl.BlockSpec((1, c), lambda r: (0, 0)),
        pl.BlockSpec((1, c), lambda r: (0, 0)),
    ]
    if res is not None:
        ins.append(res)
        specs.append(pl.BlockSpec((BM, c), lambda r: (r, 0)))

    def body(*refs):
        x_ref, s_ref, g_ref, b_ref = refs[:4]
        o_ref = refs[-1]
        m = s_ref[0:1, :] * (1.0 / N)
        var = s_ref[1:2, :] * (1.0 / N) - m * m
        y = (x_ref[...] - m) * lax.rsqrt(var + 1e-5) * g_ref[...] + b_ref[...]
        if res is not None:
            y = y + refs[4][...]
        if act == "elu":
            y = jnp.where(y > 0, y, jnp.exp(jnp.minimum(y, 0.0)) - 1.0)
        o_ref[...] = y

    return pl.pallas_call(
        body,
        grid=(N // BM,),
        in_specs=specs,
        out_specs=pl.BlockSpec((BM, c), lambda r: (r, 0)),
        out_shape=jax.ShapeDtypeStruct((N, c), jnp.float32),
    )(*ins)


def _head(hv8, hp8, w1, b1, gam, bet, w2, b2):
    def body(a_ref, p_ref, w1_ref, b1_ref, g_ref, be_ref, w2_ref, b2_ref,
             o_ref):
        x = jnp.concatenate([a_ref[...], p_ref[...]], axis=1)
        h = jnp.dot(x, w1_ref[...], preferred_element_type=jnp.float32) \
            + b1_ref[...]
        h = jnp.where(h > 0, h, jnp.exp(jnp.minimum(h, 0.0)) - 1.0)
        m = jnp.mean(h, 0, keepdims=True)
        v = jnp.mean(h * h, 0, keepdims=True) - m * m
        h = (h - m) * lax.rsqrt(v + 1e-5) * g_ref[...] + be_ref[...]
        z = jnp.dot(h, w2_ref[...], preferred_element_type=jnp.float32) \
            + b2_ref[...]
        mx = jnp.max(z, 1, keepdims=True)
        lse = jnp.log(jnp.sum(jnp.exp(z - mx), 1, keepdims=True)) + mx
        o_ref[...] = z - lse

    nclass = w2.shape[1]
    return pl.pallas_call(
        body,
        out_shape=jax.ShapeDtypeStruct((8, nclass), jnp.float32),
    )(hv8, hp8, w1, b1.reshape(1, -1), gam.reshape(1, -1),
      bet.reshape(1, -1), w2, b2.reshape(1, -1))


@functools.cache
def _pair_kernel():
    mesh = plsc.VectorSubcoreMesh(core_axis_name="c", subcore_axis_name="s")

    @functools.partial(
        pl.kernel,
        mesh=mesh,
        out_type=jax.ShapeDtypeStruct((4 * NP, 128), jnp.float32),
        scratch_types=[
            pltpu.VMEM((SB // B, B), jnp.int32),
            pltpu.VMEM((SB // B, B), jnp.int32),
            pltpu.VMEM((SB * 16,), jnp.float32),
            pltpu.VMEM((2, B, 640), jnp.float32),
            pltpu.VMEM((2, B, 128), jnp.float32),
            pltpu.VMEM_SHARED((NP, 128), jnp.float32),
            pltpu.SemaphoreType.DMA,
        ],
    )
    def kern(xg5, xg1, src2, dst2, gaussr, zr, out, srcv2, dstv2,
             gaussv, gbuf, msg, spmem, sem):
        p = lax.axis_index("c")
        s = lax.axis_index("s")
        wid = s * NC + p
        ebase = wid * EPT
        nsb = SB // B
        for ph in range(2):
            pltpu.sync_copy(zr, spmem.at[pl.ds(s * RPT, RPT)])
            plsc.subcore_barrier()

            def sbody(sb, carry):
                e0 = pl.multiple_of(ebase + sb * SB, SB)
                r0 = pl.multiple_of(e0 // B, SB // B)
                pltpu.sync_copy(src2.at[pl.ds(r0, nsb)], srcv2)
                pltpu.sync_copy(dst2.at[pl.ds(r0, nsb)], dstv2)
                g0 = pl.multiple_of(e0 * 16, SB * 16)
                pltpu.sync_copy(gaussr.at[pl.ds(g0, SB * 16)], gaussv)

                def gather(j, bank):
                    tab = xg5 if ph == 0 else xg1
                    dst = gbuf.at[bank] if ph == 0 else msg.at[bank]
                    return pltpu.async_copy(tab.at[srcv2.at[j]], dst, sem)

                gather(0, 0).wait()

                def jbody(j, c1):
                    bank = lax.rem(j, 2)
                    nbank = lax.rem(j + 1, 2)
                    jn = lax.min(j + 1, nsb - 1)
                    d = gather(jn, nbank)

                    def ebody(e, c2):
                        gv = gaussv[pl.ds((j * B + e) * 16, 16)]
                        for jj in range(8):
                            sl = pl.ds(jj * 16, 16)
                            if ph == 0:
                                acc = gv[0] * gbuf[bank, e, sl]
                                for kk in range(1, 5):
                                    acc = acc + gv[kk] * gbuf[
                                        bank, e, pl.ds(kk * 128 + jj * 16, 16)]
                                msg[bank, e, sl] = acc
                            else:
                                msg[bank, e, sl] = gv[5] * msg[bank, e, sl]
                        return c2

                    lax.fori_loop(0, B, ebody, 0)
                    pltpu.sync_copy(msg.at[bank], spmem.at[dstv2.at[j]],
                                    add=True)
                    d.wait()
                    return c1

                lax.fori_loop(0, nsb, jbody, 0)
                return carry

            lax.fori_loop(0, EPT // SB, sbody, 0)
            plsc.subcore_barrier()
            off = (ph * 2 + p) * NP + s * RPT
            pltpu.sync_copy(spmem.at[pl.ds(s * RPT, RPT)],
                            out.at[pl.ds(off, RPT)])
            plsc.subcore_barrier()

    return kern


def _pair_call(xg5, xg1, srcp, dstp, g16):
    zr = jnp.zeros((RPT, 128), jnp.float32)
    src2 = srcp.reshape(EP // B, B)
    dst2 = dstp.reshape(EP // B, B)
    out = _pair_kernel()(xg5, xg1, src2, dst2, g16.reshape(EP * 16), zr)
    return out.reshape(2, 2, NP, 128)


def _dummy1():
    return jnp.zeros((N, 128), jnp.float32)


def _xg_chunks(x, p, k):
    cin, cout = p["root"].shape
    c = min(cout, 128)
    nch = cout // c
    g4 = p["g"].reshape(cin, k, nch, c)
    if c != 128:
        g4 = jnp.pad(g4, ((0, 0), (0, 0), (0, 0), (0, 128 - c)))
    wp = g4.transpose(0, 2, 1, 3).reshape(cin, nch * k * 128)
    xgc = _mm_xg(x, wp, nch, k * 128)
    tabs = [xgc[ch] for ch in range(nch)]
    return tabs, nch, c


def _conv_from_aggs(aggs, cntp, x, p, nch, c):
    cin = p["root"].shape[0]
    aggf = jnp.stack(aggs, axis=1)
    rootp = p["root"].reshape(cin, nch, c)
    biasp = p["bias"].reshape(1, nch, c)
    return _conv_out(aggf, cntp, x, rootp, biasp, nch, c, 128)


def _gmm_conv5(x, srcp, dstp, g16, cntp, p):
    tabs, nch, c = _xg_chunks(x, p, 5)
    aggs = [_pair_call(t, _dummy1(), srcp, dstp, g16)[0] for t in tabs]
    return _conv_from_aggs(aggs, cntp, x, p, nch, c)


def _gauss_weights(plist):
    w1, w2, bb = [], [], []
    for p in plist:
        mu, sigma = p["mu"], p["sigma"]
        inv = 1.0 / (1e-15 + sigma * sigma)
        w1.append((-0.5 * inv).T)
        w2.append((mu * inv).T)
        bb.append(-0.5 * jnp.sum(mu * mu * inv, axis=1))
    return (jnp.concatenate(w1, 1), jnp.concatenate(w2, 1),
            jnp.concatenate(bb)[None, :])


def _g16(cols):
    return jnp.pad(cols, ((0, 0), (0, 16 - cols.shape[1])))


def _res_block(x, srcp, dstp, g_a, g_c2, cntp, p):
    tabs1, nch, c = _xg_chunks(x, p["c1"], 5)
    tabs_s, _, _ = _xg_chunks(x, p["sc"], 1)
    agg_a, agg_b = [], []
    for t5, t1 in zip(tabs1, tabs_s):
        o4 = _pair_call(t5, t1, srcp, dstp, g_a)
        agg_a.append(o4[0])
        agg_b.append(o4[1])
    o1 = _conv_from_aggs(agg_a, cntp, x, p["c1"], nch, c)
    os = _conv_from_aggs(agg_b, cntp, x, p["sc"], nch, c)
    h = _bn_apply(o1, p["bn1"], act="elu")
    o2 = _gmm_conv5(h, srcp, dstp, g_c2, cntp, p["c2"])
    sbn = _bn_apply(os, p["bns"])
    return _bn_apply(o2, p["bn2"], res=sbn, act="elu")


def _stream(x0, ei, ea, p_conv, p_bn, blocks):
    srcp = jnp.pad(ei[0], (0, EP - E)).astype(jnp.int32)
    dstp = jnp.pad(ei[1], (0, EP - E)).astype(jnp.int32)
    plist = [p_conv]
    for b in blocks:
        plist += [b["c1"], b["c2"], b["sc"]]
    w1, w2, bb = _gauss_weights(plist)
    gall = _gauss_all(ea, w1, w2, bb)
    gpad = jnp.pad(gall, ((0, EP - E), (0, 0)))
    valid = (jnp.arange(EP) < E).astype(jnp.float32)[:, None]
    g_conv = _g16(jnp.concatenate([gpad[:, 0:5], valid], axis=1))
    ones_tab = jnp.ones((N, 128), jnp.float32)
    tabs0, nch0, c0 = _xg_chunks(x0, p_conv, 5)
    o4 = _pair_call(tabs0[0], ones_tab, srcp, dstp, g_conv)
    cntp = o4[1]
    o0 = _conv_from_aggs([o4[0]], cntp, x0, p_conv, nch0, c0)
    h = _bn_apply(o0, p_bn, act="elu")
    goff = 5
    for b in blocks:
        g_a = _g16(jnp.concatenate(
            [gpad[:, goff:goff + 5], gpad[:, goff + 10:goff + 11]], axis=1))
        g_c2 = _g16(gpad[:, goff + 5:goff + 10])
        h = _res_block(h, srcp, dstp, g_a, g_c2, cntp, b)
        goff += 11
    return h


def kernel(x_v, edge_index_v, edge_attr_v, ptr_v, x_p, edge_index_p,
           edge_attr_p, ptr_p, params):
    pr = params
    hv = _stream(x_v, edge_index_v, edge_attr_v, pr["conv_v"], pr["bn_v"],
                 [pr["block1"], pr["block2"], pr["block3"]])
    hp = _stream(x_p, edge_index_p, edge_attr_p, pr["conv_p"], pr["bn_p"],
                 [pr["block4"], pr["block5"], pr["block6"]])
    vi_v = ptr_v[1:] - 1
    vi_p = ptr_p[1:] - 1
    return _head(hv[vi_v], hp[vi_p], pr["fc1_w"], pr["fc1_b"],
                 pr["bn_fc"]["gamma"], pr["bn_fc"]["beta"],
                 pr["fc2_w"], pr["fc2_b"])
